# baseline (speedup 1.0000x reference)
import functools
import math

import jax
import jax.numpy as jnp
from jax.experimental import pallas as pl
from jax.experimental.pallas import tpu as pltpu
from jax.experimental.pallas import tpu_sc as plsc

S, B, H, D = 4096, 2, 2048, 64
NGRAMS = (4, 6, 8)
ND = 3 * D
W = 16
EPS = 1e-5

CHUNK = 256
HALO = 16
G = CHUNK + HALO
GCH = ((0, 128), (128, 128), (256, 16))
POW5 = (1, 5, 25, 125, 625, 3125, 15625, 78125)


def _sc_body(ii, t4, t6, t8, feat, tv, x4, x6, x8, r4, r6, r8, pooled,
             s4, s6, s8):
    cid = jax.lax.axis_index("c")
    sid = jax.lax.axis_index("s")
    wid = sid * 2 + cid
    b0 = wid // 16
    s0 = (wid % 16) * CHUNK

    lane = jnp.arange(16, dtype=jnp.int32)
    zi = jnp.zeros((16,), jnp.int32)
    zf = jnp.zeros((16,), jnp.float32)

    tv[pl.ds(0, 16)] = zi
    tv[pl.ds(16, 16)] = zi

    @pl.when(s0 == 0)
    def _():
        st = pl.multiple_of(b0 * S, 8)
        pltpu.sync_copy(ii.at[pl.ds(st, CHUNK)], tv.at[pl.ds(24, CHUNK)])

    @pl.when(s0 != 0)
    def _():
        st = pl.multiple_of(b0 * S + s0 - 24, 8)
        pltpu.sync_copy(ii.at[pl.ds(st, CHUNK + 24)],
                        tv.at[pl.ds(0, CHUNK + 24)])

    for gi in range(G // 16):
        c, off = divmod(gi * 16, 128)
        toks = [tv[pl.ds(gi * 16 + 8 - k, 16)] for k in range(8)]
        a4 = toks[0] + toks[1] * 5 + toks[2] * 25 + toks[3] * 125
        a6 = a4 + toks[4] * 625 + toks[5] * 3125
        a8 = a6 + toks[6] * 15625 + toks[7] * 78125
        x4[c, pl.ds(off, 16)] = a4
        x6[c, pl.ds(off, 16)] = a6
        x8[c, pl.ds(off, 16)] = a8

    waits = []
    for x, r, t, sem in ((x4, r4, t4, s4), (x6, r6, t6, s6), (x8, r8, t8, s8)):
        ds = []
        for ci, (o, ln) in enumerate(GCH):
            if ln == 128:
                idx = x.at[ci]
            else:
                idx = x.at[ci, pl.ds(0, ln)]
            ds.append(pltpu.async_copy(t.at[idx], r.at[pl.ds(o, ln)], sem))
        waits.append(ds)

    for g, (n, rref, dlist) in enumerate(zip(NGRAMS, (r4, r6, r8), waits)):
        L = W - n + 1
        for dsc in dlist:
            dsc.wait()

        @pl.when(s0 == 0)
        def _(rref=rref, n=n):
            for p in range(HALO + n - 1):
                for q in range(4):
                    rref[p, pl.ds(q * 16, 16)] = zf

        u = [zf, zf, zf, zf]
        for p in range(HALO - L, HALO):
            for q in range(4):
                u[q] = u[q] + rref[p, pl.ds(q * 16, 16)]

        def body(i, u, rref=rref, L=L, g=g):
            out = []
            for q, uq in enumerate(u):
                uq = uq + rref[i + HALO, pl.ds(q * 16, 16)]
                uq = uq - rref[i + HALO - L, pl.ds(q * 16, 16)]
                pooled[i, pl.ds(g * D + q * 16, 16)] = uq
                out.append(uq)
            return tuple(out)

        jax.lax.fori_loop(0, CHUNK, body, tuple(u))

    pltpu.sync_copy(pooled, feat.at[b0, pl.ds(s0, CHUNK)])


def _sc_feat(input_ids, emb4, emb6, emb8):
    mesh = plsc.VectorSubcoreMesh(core_axis_name="c", subcore_axis_name="s")
    fn = pl.kernel(
        _sc_body,
        out_type=jax.ShapeDtypeStruct((B, S, ND), jnp.float32),
        mesh=mesh,
        scratch_types=[
            pltpu.VMEM((320,), jnp.int32),
            pltpu.VMEM((3, 128), jnp.int32),
            pltpu.VMEM((3, 128), jnp.int32),
            pltpu.VMEM((3, 128), jnp.int32),
            pltpu.VMEM((G, D), jnp.float32),
            pltpu.VMEM((G, D), jnp.float32),
            pltpu.VMEM((G, D), jnp.float32),
            pltpu.VMEM((CHUNK, ND), jnp.float32),
            pltpu.SemaphoreType.DMA,
            pltpu.SemaphoreType.DMA,
            pltpu.SemaphoreType.DMA,
        ],
        compiler_params=pltpu.CompilerParams(use_tc_tiling_on_sc=False),
    )
    return fn(input_ids.reshape(-1), emb4, emb6, emb8)


def _tc_body(h_ref, f_ref, kw_ref, vw_ref, nq_ref, nk_ref, o_ref):
    kw = kw_ref[...]
    vw = vw_ref[...]
    nq = nq_ref[...]
    nk = nk_ref[...]
    dn = (((1,), (1,)), ((), ()))
    inv_sqrt_h = 1.0 / math.sqrt(float(H))
    bs = h_ref.shape[0]
    trow = (jax.lax.broadcasted_iota(jnp.int32, (bs, ND), 0)
            + pl.program_id(0) * bs)
    ncol = 4 + 2 * (jax.lax.broadcasted_iota(jnp.int32, (bs, ND), 1) // D)
    dnm = jnp.clip(trow - ncol + 2, 1, W - ncol + 1).astype(jnp.float32)
    rcp = 1.0 / dnm
    for b in range(B):
        h = h_ref[:, b, :]
        f = f_ref[b] * rcp
        k = jax.lax.dot_general(f, kw, dn, precision=jax.lax.Precision.DEFAULT,
                                preferred_element_type=jnp.float32)
        v = jax.lax.dot_general(f, vw, dn, precision=jax.lax.Precision.DEFAULT,
                                preferred_element_type=jnp.float32)
        q = h * jax.lax.rsqrt(jnp.mean(h * h, axis=-1, keepdims=True) + EPS) * nq
        kn = k * jax.lax.rsqrt(jnp.mean(k * k, axis=-1, keepdims=True) + EPS) * nk
        x = jnp.sum(q * kn, axis=-1, keepdims=True) * inv_sqrt_h
        gate = 1.0 / (1.0 + jnp.exp(-x))
        o_ref[:, b, :] = gate * v


def _tc_call(hidden, feat, key_w, value_w, nq, nk, interpret=False):
    BS = 512
    return pl.pallas_call(
        _tc_body,
        grid=(S // BS,),
        in_specs=[
            pl.BlockSpec((BS, B, H), lambda i: (i, 0, 0)),
            pl.BlockSpec((B, BS, ND), lambda i: (0, i, 0)),
            pl.BlockSpec((H, ND), lambda i: (0, 0)),
            pl.BlockSpec((H, ND), lambda i: (0, 0)),
            pl.BlockSpec((1, H), lambda i: (0, 0)),
            pl.BlockSpec((1, H), lambda i: (0, 0)),
        ],
        out_specs=pl.BlockSpec((BS, B, H), lambda i: (i, 0, 0)),
        out_shape=jax.ShapeDtypeStruct((S, B, H), jnp.float32),
        compiler_params=pltpu.CompilerParams(
            vmem_limit_bytes=128 * 1024 * 1024),
        interpret=interpret,
    )(hidden, feat, key_w, value_w, nq, nk)


def kernel(hidden_states, input_ids, emb4, emb6, emb8, key_w, value_w,
           normq_w, normk_w):
    feat = _sc_feat(input_ids, emb4, emb6, emb8)
    return _tc_call(hidden_states, feat, key_w, value_w,
                    normq_w.reshape(1, H), normk_w.reshape(1, H))


# speedup vs baseline: 1.0018x; 1.0018x over previous
import functools
import math

import jax
import jax.numpy as jnp
from jax.experimental import pallas as pl
from jax.experimental.pallas import tpu as pltpu
from jax.experimental.pallas import tpu_sc as plsc

S, B, H, D = 4096, 2, 2048, 64
NGRAMS = (4, 6, 8)
ND = 3 * D
W = 16
EPS = 1e-5

CHUNK = 256
HALO = 16
G = CHUNK + HALO
GCH = ((0, 128), (128, 128), (256, 16))
POW5 = (1, 5, 25, 125, 625, 3125, 15625, 78125)


def _sc_body(ii, t4, t6, t8, feat, tv, x4, x6, x8, r4, r6, r8, pooled,
             s4, s6, s8):
    cid = jax.lax.axis_index("c")
    sid = jax.lax.axis_index("s")
    wid = sid * 2 + cid
    b0 = wid // 16
    s0 = (wid % 16) * CHUNK

    lane = jnp.arange(16, dtype=jnp.int32)
    zi = jnp.zeros((16,), jnp.int32)
    zf = jnp.zeros((16,), jnp.float32)

    tv[pl.ds(0, 16)] = zi
    tv[pl.ds(16, 16)] = zi

    @pl.when(s0 == 0)
    def _():
        st = pl.multiple_of(b0 * S, 8)
        pltpu.sync_copy(ii.at[pl.ds(st, CHUNK)], tv.at[pl.ds(24, CHUNK)])

    @pl.when(s0 != 0)
    def _():
        st = pl.multiple_of(b0 * S + s0 - 24, 8)
        pltpu.sync_copy(ii.at[pl.ds(st, CHUNK + 24)],
                        tv.at[pl.ds(0, CHUNK + 24)])

    for gi in range(G // 16):
        c, off = divmod(gi * 16, 128)
        toks = [tv[pl.ds(gi * 16 + 8 - k, 16)] for k in range(8)]
        a4 = toks[0] + toks[1] * 5 + toks[2] * 25 + toks[3] * 125
        a6 = a4 + toks[4] * 625 + toks[5] * 3125
        a8 = a6 + toks[6] * 15625 + toks[7] * 78125
        x4[c, pl.ds(off, 16)] = a4
        x6[c, pl.ds(off, 16)] = a6
        x8[c, pl.ds(off, 16)] = a8

    waits = []
    for x, r, t, sem in ((x4, r4, t4, s4), (x6, r6, t6, s6), (x8, r8, t8, s8)):
        ds = []
        for ci, (o, ln) in enumerate(GCH):
            if ln == 128:
                idx = x.at[ci]
            else:
                idx = x.at[ci, pl.ds(0, ln)]
            ds.append(pltpu.async_copy(t.at[idx], r.at[pl.ds(o, ln)], sem))
        waits.append(ds)

    for g, (n, rref, dlist) in enumerate(zip(NGRAMS, (r4, r6, r8), waits)):
        L = W - n + 1
        for dsc in dlist:
            dsc.wait()

        @pl.when(s0 == 0)
        def _(rref=rref, n=n):
            for p in range(HALO + n - 1):
                for q in range(4):
                    rref[p, pl.ds(q * 16, 16)] = zf

        u = [zf, zf, zf, zf]
        for p in range(HALO - L, HALO):
            for q in range(4):
                u[q] = u[q] + rref[p, pl.ds(q * 16, 16)]

        def body(i, u, rref=rref, L=L, g=g):
            out = []
            for q, uq in enumerate(u):
                uq = uq + rref[i + HALO, pl.ds(q * 16, 16)]
                uq = uq - rref[i + HALO - L, pl.ds(q * 16, 16)]
                pooled[i, pl.ds(g * D + q * 16, 16)] = uq
                out.append(uq)
            return tuple(out)

        jax.lax.fori_loop(0, CHUNK, body, tuple(u))

    pltpu.sync_copy(pooled, feat.at[b0, pl.ds(s0, CHUNK)])


def _sc_feat(input_ids, emb4, emb6, emb8):
    mesh = plsc.VectorSubcoreMesh(core_axis_name="c", subcore_axis_name="s")
    fn = pl.kernel(
        _sc_body,
        out_type=jax.ShapeDtypeStruct((B, S, ND), jnp.float32),
        mesh=mesh,
        scratch_types=[
            pltpu.VMEM((320,), jnp.int32),
            pltpu.VMEM((3, 128), jnp.int32),
            pltpu.VMEM((3, 128), jnp.int32),
            pltpu.VMEM((3, 128), jnp.int32),
            pltpu.VMEM((G, D), jnp.float32),
            pltpu.VMEM((G, D), jnp.float32),
            pltpu.VMEM((G, D), jnp.float32),
            pltpu.VMEM((CHUNK, ND), jnp.float32),
            pltpu.SemaphoreType.DMA,
            pltpu.SemaphoreType.DMA,
            pltpu.SemaphoreType.DMA,
        ],
        compiler_params=pltpu.CompilerParams(use_tc_tiling_on_sc=False),
    )
    return fn(input_ids.reshape(-1), emb4, emb6, emb8)


def _tc_body(h_ref, f_ref, kw_ref, vw_ref, nq_ref, nk_ref, o_ref):
    kw = kw_ref[...]
    vw = vw_ref[...]
    nq = nq_ref[...]
    nk = nk_ref[...]
    dn = (((1,), (1,)), ((), ()))
    inv_sqrt_h = 1.0 / math.sqrt(float(H))
    bs = h_ref.shape[0]
    trow = (jax.lax.broadcasted_iota(jnp.int32, (bs, ND), 0)
            + pl.program_id(0) * bs)
    ncol = 4 + 2 * (jax.lax.broadcasted_iota(jnp.int32, (bs, ND), 1) // D)
    dnm = jnp.clip(trow - ncol + 2, 1, W - ncol + 1).astype(jnp.float32)
    rcp = 1.0 / dnm
    for b in range(B):
        h = h_ref[:, b, :]
        f = f_ref[b] * rcp
        k = jax.lax.dot_general(f, kw, dn, precision=jax.lax.Precision.DEFAULT,
                                preferred_element_type=jnp.float32)
        v = jax.lax.dot_general(f, vw, dn, precision=jax.lax.Precision.DEFAULT,
                                preferred_element_type=jnp.float32)
        q = h * jax.lax.rsqrt(jnp.mean(h * h, axis=-1, keepdims=True) + EPS) * nq
        kn = k * jax.lax.rsqrt(jnp.mean(k * k, axis=-1, keepdims=True) + EPS) * nk
        x = jnp.sum(q * kn, axis=-1, keepdims=True) * inv_sqrt_h
        gate = 1.0 / (1.0 + jnp.exp(-x))
        o_ref[:, b, :] = gate * v


def _tc_call(hidden, feat, key_w, value_w, nq, nk, interpret=False):
    BS = 256
    return pl.pallas_call(
        _tc_body,
        grid=(S // BS,),
        in_specs=[
            pl.BlockSpec((BS, B, H), lambda i: (i, 0, 0)),
            pl.BlockSpec((B, BS, ND), lambda i: (0, i, 0)),
            pl.BlockSpec((H, ND), lambda i: (0, 0)),
            pl.BlockSpec((H, ND), lambda i: (0, 0)),
            pl.BlockSpec((1, H), lambda i: (0, 0)),
            pl.BlockSpec((1, H), lambda i: (0, 0)),
        ],
        out_specs=pl.BlockSpec((BS, B, H), lambda i: (i, 0, 0)),
        out_shape=jax.ShapeDtypeStruct((S, B, H), jnp.float32),
        compiler_params=pltpu.CompilerParams(
            vmem_limit_bytes=128 * 1024 * 1024),
        interpret=interpret,
    )(hidden, feat, key_w, value_w, nq, nk)


def kernel(hidden_states, input_ids, emb4, emb6, emb8, key_w, value_w,
           normq_w, normk_w):
    feat = _sc_feat(input_ids, emb4, emb6, emb8)
    return _tc_call(hidden_states, feat, key_w, value_w,
                    normq_w.reshape(1, H), normk_w.reshape(1, H))
